# baseline (speedup 1.0000x reference)
import jax
import jax.numpy as jnp
from jax import lax
from jax.experimental import pallas as pl
from jax.experimental.pallas import tpu as pltpu
from jax.experimental.pallas import tpu_sc as plsc

VOCAB = 1000
N_TOK = 51200
NC, NS = 2, 16
NW = NC * NS
LSE_PAD = 1024

N_VT = VOCAB // 8
PC = 2560
N_PT = PC // 128
N_CH = N_TOK // PC
VT_PER_W = (N_VT + NW - 1) // NW

LW = N_TOK // NW
LG = LW // 16
LD = 80
N_LD = LW // LD


def _prep_body(x_ref, lse_ref, tt_ref):
    x = x_ref[...]
    m = jnp.max(x, axis=1)
    s = jnp.sum(jnp.exp(x - m[:, None]), axis=1)
    lse = m + jnp.log(s)
    lse_ref[...] = jnp.concatenate(
        [lse, jnp.zeros((LSE_PAD - VOCAB,), jnp.float32)]
    )[:, None]
    tt_ref[...] = x.T


@jax.jit
def _prep_call(table):
    return pl.pallas_call(
        _prep_body,
        out_shape=(
            jax.ShapeDtypeStruct((LSE_PAD, 1), jnp.float32),
            jax.ShapeDtypeStruct((VOCAB, VOCAB), jnp.float32),
        ),
    )(table)


def _sc_body(tt, table1m, idxf, tf, lse, out4, partials,
             slab, idx_v, buf, lse_v, idxl_v, tl_v, lin_v, vals_v, acc,
             semi, sems, semv):
    c_id = lax.axis_index("c")
    s_id = lax.axis_index("s")
    wid = s_id * NC + c_id

    def out_desc(tc, j, b):
        return pltpu.make_async_copy(
            buf.at[b], out4.at[tc, pl.ds(j * N_PT, N_PT)], sems.at[b]
        )

    for k in range(VT_PER_W):
        tc = wid + NW * k

        @pl.when(tc < N_VT)
        def _tile_row():
            pltpu.sync_copy(tt.at[pl.ds(tc * 8, 8)], slab)

            for j in range(N_CH):
                b = j % 2
                if j >= 2:
                    out_desc(tc, j - 2, b).wait()
                pltpu.sync_copy(idxf.at[pl.ds(j * PC, PC)], idx_v)

                def group(m, carry):
                    iv = idx_v[pl.ds(m * 16, 16)]
                    t = m // 8
                    l0 = (m % 8) * 16
                    for s in range(8):
                        val = plsc.load_gather(
                            slab, [jnp.full((16,), s, jnp.int32), iv]
                        )
                        buf[b, t, s, pl.ds(l0, 16)] = val
                    return carry

                lax.fori_loop(0, PC // 16, group, 0)
                out_desc(tc, j, b).start()
            out_desc(tc, N_CH - 2, 0).wait()
            out_desc(tc, N_CH - 1, 1).wait()

    base = wid * LW
    pltpu.sync_copy(lse, lse_v)
    pltpu.sync_copy(idxf.at[pl.ds(base, LW)], idxl_v)
    pltpu.sync_copy(tf.at[pl.ds(base, LW)], tl_v)

    def build_lin(m, carry):
        iv = idxl_v[pl.ds(m * 16, 16)]
        tv = tl_v[pl.ds(m * 16, 16)]
        lin_v[m // 5, pl.ds((m % 5) * 16, 16)] = iv * VOCAB + tv
        return carry

    lax.fori_loop(0, LG, build_lin, 0)

    for d in range(N_LD):
        pltpu.async_copy(table1m.at[lin_v.at[d]], vals_v.at[d], semv)
    for d in range(N_LD):
        pltpu.make_async_copy(
            table1m.at[lin_v.at[d]], vals_v.at[d], semv
        ).wait()

    acc[...] = jnp.zeros((16,), jnp.float32)
    zeros16 = jnp.zeros((16,), jnp.int32)
    ios = lax.iota(jnp.int32, 16)

    def accum(m, carry):
        iv = idxl_v[pl.ds(m * 16, 16)]
        d = m // 5
        o = (m % 5) * 16
        vals = plsc.load_gather(vals_v, [jnp.full((16,), 1, jnp.int32) * d,
                                         o + ios, zeros16])
        lsev = plsc.load_gather(lse_v, [iv])
        acc[...] = acc[...] + (lsev - vals)
        return carry

    lax.fori_loop(0, LG, accum, 0)
    pltpu.sync_copy(acc, partials.at[wid])


@jax.jit
def _sc_call(tt, table1m, idx_f, t_f, lse_flat):
    mesh = plsc.VectorSubcoreMesh(
        core_axis_name="c", subcore_axis_name="s", num_cores=NC,
        num_subcores=NS,
    )
    return pl.kernel(
        _sc_body,
        out_type=(
            jax.ShapeDtypeStruct((N_VT, N_TOK // 128, 8, 128), jnp.float32),
            jax.ShapeDtypeStruct((NW, 16), jnp.float32),
        ),
        mesh=mesh,
        compiler_params=pltpu.CompilerParams(
            use_tc_tiling_on_sc=False, needs_layout_passes=False
        ),
        scratch_types=[
            pltpu.VMEM((8, VOCAB), jnp.float32),
            pltpu.VMEM((PC,), jnp.int32),
            pltpu.VMEM((2, N_PT, 8, 128), jnp.float32),
            pltpu.VMEM((LSE_PAD,), jnp.float32),
            pltpu.VMEM((LW,), jnp.int32),
            pltpu.VMEM((LW,), jnp.int32),
            pltpu.VMEM((N_LD, LD), jnp.int32),
            pltpu.VMEM((N_LD, LD, 1), jnp.float32),
            pltpu.VMEM((16,), jnp.float32),
            pltpu.SemaphoreType.DMA,
            pltpu.SemaphoreType.DMA((2,)),
            pltpu.SemaphoreType.DMA,
        ],
    )(tt, table1m, idx_f, t_f, lse_flat)


def _loss_body(p_ref, o_ref):
    o_ref[...] = (jnp.sum(p_ref[...]) / N_TOK).reshape(1, 1)


@jax.jit
def _loss_call(partials):
    return pl.pallas_call(
        _loss_body,
        out_shape=jax.ShapeDtypeStruct((1, 1), jnp.float32),
    )(partials)


def kernel(idx, targets, token_emb):
    idx_f = idx.reshape(-1).astype(jnp.int32)
    t_f = targets.reshape(-1).astype(jnp.int32)
    lse, tt = _prep_call(token_emb)
    out4, partials = _sc_call(
        tt, token_emb.reshape(VOCAB * VOCAB, 1), idx_f, t_f,
        lse.reshape(LSE_PAD),
    )
    logits2 = out4.transpose(1, 3, 0, 2).reshape(N_TOK, VOCAB)
    loss = _loss_call(partials)[0, 0]
    return logits2, loss


# speedup vs baseline: 1.3173x; 1.3173x over previous
import jax
import jax.numpy as jnp
from jax import lax
from jax.experimental import pallas as pl
from jax.experimental.pallas import tpu as pltpu
from jax.experimental.pallas import tpu_sc as plsc

VOCAB = 1000
N_TOK = 51200
NC, NS = 2, 16
NW = NC * NS
LSE_PAD = 1024

N_VT = VOCAB // 8
PC = 2560
N_PT = PC // 128
N_CH = N_TOK // PC
VT_PER_W = (N_VT + NW - 1) // NW

LW = N_TOK // NW
LG = LW // 16
LD = 80
N_LD = LW // LD


def _prep_body(x_ref, lse_ref, tt_ref):
    x = x_ref[...]
    m = jnp.max(x, axis=1)
    s = jnp.sum(jnp.exp(x - m[:, None]), axis=1)
    lse = m + jnp.log(s)
    lse_ref[...] = jnp.concatenate(
        [lse, jnp.zeros((LSE_PAD - VOCAB,), jnp.float32)]
    )[:, None]
    tt_ref[...] = x.T


@jax.jit
def _prep_call(table):
    return pl.pallas_call(
        _prep_body,
        out_shape=(
            jax.ShapeDtypeStruct((LSE_PAD, 1), jnp.float32),
            jax.ShapeDtypeStruct((VOCAB, VOCAB), jnp.float32),
        ),
    )(table)


def _sc_body(tt, table1m, idxf, tf, lse, out4, partials,
             slab, idx_v, buf, lse_v, idxl_v, tl_v, lin_v, vals_v, acc,
             semi, sems, semv):
    c_id = lax.axis_index("c")
    s_id = lax.axis_index("s")
    wid = s_id * NC + c_id

    def out_desc(tc, j, b):
        return pltpu.make_async_copy(
            buf.at[b], out4.at[tc, pl.ds(j * N_PT, N_PT)], sems.at[b]
        )

    for k in range(VT_PER_W):
        tc = wid + NW * k

        @pl.when(tc < N_VT)
        def _tile_row():
            pltpu.sync_copy(tt.at[pl.ds(tc * 8, 8)], slab)

            def chunk_pair(q, carry):
                for b in range(2):
                    j = 2 * q + b

                    @pl.when(j >= 2)
                    def _drain():
                        out_desc(tc, j - 2, b).wait()

                    pltpu.sync_copy(idxf.at[pl.ds(j * PC, PC)], idx_v)

                    @plsc.parallel_loop(0, PC // 16, unroll=4)
                    def group(m):
                        iv = idx_v[pl.ds(m * 16, 16)]
                        t = m // 8
                        l0 = (m % 8) * 16
                        vals = [
                            plsc.load_gather(
                                slab, [jnp.full((16,), s, jnp.int32), iv]
                            )
                            for s in range(8)
                        ]
                        for s in range(8):
                            buf[b, t, s, pl.ds(l0, 16)] = vals[s]
                    out_desc(tc, j, b).start()
                return carry

            lax.fori_loop(0, N_CH // 2, chunk_pair, 0)
            out_desc(tc, N_CH - 2, 0).wait()
            out_desc(tc, N_CH - 1, 1).wait()

    base = wid * LW
    pltpu.sync_copy(lse, lse_v)
    pltpu.sync_copy(idxf.at[pl.ds(base, LW)], idxl_v)
    pltpu.sync_copy(tf.at[pl.ds(base, LW)], tl_v)

    @plsc.parallel_loop(0, LG, unroll=4)
    def build_lin(m):
        iv = idxl_v[pl.ds(m * 16, 16)]
        tv = tl_v[pl.ds(m * 16, 16)]
        lin_v[m // 5, pl.ds((m % 5) * 16, 16)] = iv * VOCAB + tv

    for d in range(N_LD):
        pltpu.async_copy(table1m.at[lin_v.at[d]], vals_v.at[d], semv)
    for d in range(N_LD):
        pltpu.make_async_copy(
            table1m.at[lin_v.at[d]], vals_v.at[d], semv
        ).wait()

    acc[...] = jnp.zeros((16,), jnp.float32)
    zeros16 = jnp.zeros((16,), jnp.int32)
    ios = lax.iota(jnp.int32, 16)

    def accum(m, carry):
        iv = idxl_v[pl.ds(m * 16, 16)]
        d = m // 5
        o = (m % 5) * 16
        vals = plsc.load_gather(vals_v, [jnp.full((16,), 1, jnp.int32) * d,
                                         o + ios, zeros16])
        lsev = plsc.load_gather(lse_v, [iv])
        acc[...] = acc[...] + (lsev - vals)
        return carry

    lax.fori_loop(0, LG, accum, 0)
    pltpu.sync_copy(acc, partials.at[wid])


@jax.jit
def _sc_call(tt, table1m, idx_f, t_f, lse_flat):
    mesh = plsc.VectorSubcoreMesh(
        core_axis_name="c", subcore_axis_name="s", num_cores=NC,
        num_subcores=NS,
    )
    return pl.kernel(
        _sc_body,
        out_type=(
            jax.ShapeDtypeStruct((N_VT, N_TOK // 128, 8, 128), jnp.float32),
            jax.ShapeDtypeStruct((NW, 16), jnp.float32),
        ),
        mesh=mesh,
        compiler_params=pltpu.CompilerParams(
            use_tc_tiling_on_sc=False, needs_layout_passes=False
        ),
        scratch_types=[
            pltpu.VMEM((8, VOCAB), jnp.float32),
            pltpu.VMEM((PC,), jnp.int32),
            pltpu.VMEM((2, N_PT, 8, 128), jnp.float32),
            pltpu.VMEM((LSE_PAD,), jnp.float32),
            pltpu.VMEM((LW,), jnp.int32),
            pltpu.VMEM((LW,), jnp.int32),
            pltpu.VMEM((N_LD, LD), jnp.int32),
            pltpu.VMEM((N_LD, LD, 1), jnp.float32),
            pltpu.VMEM((16,), jnp.float32),
            pltpu.SemaphoreType.DMA,
            pltpu.SemaphoreType.DMA((2,)),
            pltpu.SemaphoreType.DMA,
        ],
    )(tt, table1m, idx_f, t_f, lse_flat)


def _loss_body(p_ref, o_ref):
    o_ref[...] = (jnp.sum(p_ref[...]) / N_TOK).reshape(1, 1)


@jax.jit
def _loss_call(partials):
    return pl.pallas_call(
        _loss_body,
        out_shape=jax.ShapeDtypeStruct((1, 1), jnp.float32),
    )(partials)


def kernel(idx, targets, token_emb):
    idx_f = idx.reshape(-1).astype(jnp.int32)
    t_f = targets.reshape(-1).astype(jnp.int32)
    lse, tt = _prep_call(token_emb)
    out4, partials = _sc_call(
        tt, token_emb.reshape(VOCAB * VOCAB, 1), idx_f, t_f,
        lse.reshape(LSE_PAD),
    )
    logits2 = out4.transpose(1, 3, 0, 2).reshape(N_TOK, VOCAB)
    loss = _loss_call(partials)[0, 0]
    return logits2, loss
